# baseline (speedup 1.0000x reference)
import functools

import jax
import jax.numpy as jnp
from jax import lax
from jax.experimental import pallas as pl
from jax.experimental.pallas import tpu as pltpu
from jax.experimental.pallas import tpu_sc as plsc

NUM_FIELDS = 26
FIELD_DIM = 38461
D = 16
B = 4096
NNZ = 26
NC = 2
NS = 16
NW = NC * NS
ROWS_PT = B // NW
EW = 32
E_PT = ROWS_PT * EW
CH = 128
NCH = E_PT // CH
QR = 32
NQ = ROWS_PT // QR
OUT_PT = ROWS_PT * (NUM_FIELDS - 1)
XCOL = 0
FCOL = 32


@functools.partial(
    pl.kernel,
    out_type=jax.ShapeDtypeStruct((B * (NUM_FIELDS - 1) * D,), jnp.float32),
    mesh=plsc.VectorSubcoreMesh(core_axis_name="c", subcore_axis_name="s"),
    compiler_params=pltpu.CompilerParams(use_tc_tiling_on_sc=False,
                                         needs_layout_passes=False),
    scratch_types=[
        pltpu.VMEM((QR * 128,), jnp.int32),
        pltpu.VMEM((NCH, CH), jnp.int32),
        pltpu.VMEM((E_PT,), jnp.int32),
        pltpu.VMEM((E_PT, D), jnp.float32),
        pltpu.VMEM((OUT_PT * D,), jnp.float32),
        pltpu.SemaphoreType.DMA,
    ],
)
def _emb(xc_hbm, table_hbm, out_hbm, xc_v, gidx_v, d_v, rows_v, acc_v, semg):
    wid = lax.axis_index("s") * NC + lax.axis_index("c")

    iota = lax.iota(jnp.int32, 16)
    for q in range(NQ):
        pltpu.sync_copy(
            xc_hbm.at[pl.ds((wid * ROWS_PT + q * QR) * 128, QR * 128)], xc_v)
        for rq in range(QR):
            r = q * QR + rq
            for h in range(2):
                xv = xc_v[pl.ds(rq * 128 + XCOL + h * 16, 16)]
                f = xc_v[pl.ds(rq * 128 + FCOL + h * 16, 16)]
                nz = jnp.minimum(f, 1)
                gid = (xv + f * FIELD_DIM) * nz
                d = (r * (NUM_FIELDS - 1) * D) + (f - nz) * D
                e = r * EW + h * 16
                gidx_v[e // CH, pl.ds(e % CH, 16)] = gid
                d_v[pl.ds(e, 16)] = d

    copies = [
        pltpu.async_copy(table_hbm.at[gidx_v.at[j]],
                         rows_v.at[pl.ds(j * CH, CH)], semg)
        for j in range(NCH)
    ]

    zeros = jnp.zeros((16,), jnp.float32)

    def zbody(i, carry):
        acc_v[pl.ds(i * 16, 16)] = zeros
        return carry

    lax.fori_loop(0, OUT_PT, zbody, 0)

    for c in copies:
        c.wait()

    def abody(i, carry):
        dvec = d_v[pl.ds(i * 16, 16)]
        for lane in range(16):
            e = i * 16 + lane
            vals = plsc.load_gather(rows_v,
                                    [jnp.full((16,), e, jnp.int32), iota])
            plsc.addupdate_scatter(acc_v, [dvec[lane] + iota], vals)
        return carry

    lax.fori_loop(0, E_PT // 16, abody, 0)

    pltpu.sync_copy(acc_v, out_hbm.at[pl.ds(wid * OUT_PT * D, OUT_PT * D)])


def kernel(x_field, x, table):
    xf = x_field.astype(jnp.int32)
    xx = x.astype(jnp.int32)
    zc = jnp.zeros((B, 6), jnp.int32)
    zt = jnp.zeros((B, 128 - 2 * EW), jnp.int32)
    xcomb = jnp.concatenate([xx, zc, xf, zc, zt], axis=1).reshape(-1)
    out = _emb(xcomb, table)
    return out.reshape(B, NUM_FIELDS - 1, D)


# speedup vs baseline: 1.2167x; 1.2167x over previous
import functools

import jax
import jax.numpy as jnp
from jax import lax
from jax.experimental import pallas as pl
from jax.experimental.pallas import tpu as pltpu
from jax.experimental.pallas import tpu_sc as plsc

NUM_FIELDS = 26
FIELD_DIM = 38461
D = 16
B = 4096
NNZ = 26
NC = 2
NS = 16
NW = NC * NS
ROWS_PT = B // NW
E_PT = ROWS_PT * NNZ
NV = E_PT // 16
CH = 128
NCH = E_PT // CH
OUT_PT = ROWS_PT * (NUM_FIELDS - 1)


@functools.partial(
    pl.kernel,
    out_type=jax.ShapeDtypeStruct((B * (NUM_FIELDS - 1) * D,), jnp.float32),
    mesh=plsc.VectorSubcoreMesh(core_axis_name="c", subcore_axis_name="s"),
    compiler_params=pltpu.CompilerParams(use_tc_tiling_on_sc=False,
                                         needs_layout_passes=False),
    scratch_types=[
        pltpu.VMEM((E_PT,), jnp.int32),
        pltpu.VMEM((E_PT,), jnp.int32),
        pltpu.VMEM((E_PT,), jnp.int32),
        pltpu.VMEM((NCH, CH), jnp.int32),
        pltpu.VMEM((E_PT,), jnp.int32),
        pltpu.VMEM((E_PT, D), jnp.float32),
        pltpu.VMEM((OUT_PT * D,), jnp.float32),
        pltpu.SemaphoreType.DMA,
    ],
)
def _emb(xf_hbm, xx_hbm, table_hbm, dbase_hbm, out_hbm, f_v, x_v, db_v,
         gidx_v, d_v, rows_v, out_v, sem):
    wid = lax.axis_index("s") * NC + lax.axis_index("c")
    ebase = wid * E_PT
    pltpu.sync_copy(xf_hbm.at[pl.ds(ebase, E_PT)], f_v)
    pltpu.sync_copy(xx_hbm.at[pl.ds(ebase, E_PT)], x_v)
    pltpu.sync_copy(dbase_hbm, db_v)

    iota = lax.iota(jnp.int32, 16)
    for v in range(NV):
        f = f_v[pl.ds(v * 16, 16)]
        xv = x_v[pl.ds(v * 16, 16)]
        nz = jnp.minimum(f, 1)
        gid = (xv + f * FIELD_DIM) * nz
        d = (db_v[pl.ds(v * 16, 16)] + f - nz) * D
        gidx_v[v // 8, pl.ds((v % 8) * 16, 16)] = gid
        d_v[pl.ds(v * 16, 16)] = d

    copies = [
        pltpu.async_copy(table_hbm.at[gidx_v.at[j]],
                         rows_v.at[pl.ds(j * CH, CH)], sem)
        for j in range(NCH)
    ]

    zeros = jnp.zeros((16,), jnp.float32)

    def zbody(i, carry):
        out_v[pl.ds(i * 16, 16)] = zeros
        return carry

    lax.fori_loop(0, OUT_PT, zbody, 0)

    for c in copies:
        c.wait()

    def abody(i, carry):
        dvec = d_v[pl.ds(i * 16, 16)]
        for lane in range(16):
            e = i * 16 + lane
            vals = plsc.load_gather(rows_v,
                                    [jnp.full((16,), e, jnp.int32), iota])
            plsc.addupdate_scatter(out_v, [dvec[lane] + iota], vals)
        return carry

    lax.fori_loop(0, NV, abody, 0)

    pltpu.sync_copy(out_v, out_hbm.at[pl.ds(wid * OUT_PT * D, OUT_PT * D)])


def kernel(x_field, x, table):
    xf = x_field.reshape(-1).astype(jnp.int32)
    xx = x.reshape(-1).astype(jnp.int32)
    dbase = (jnp.arange(E_PT, dtype=jnp.int32) // NNZ) * (NUM_FIELDS - 1)
    out = _emb(xf, xx, table, dbase)
    return out.reshape(B, NUM_FIELDS - 1, D)
